# speedup vs baseline: 1.2084x; 1.2084x over previous
import functools

import jax
import jax.numpy as jnp
from jax import lax
from jax.experimental import pallas as pl
from jax.experimental.pallas import tpu as pltpu
from jax.experimental.pallas import tpu_sc as plsc

NUM_SAMPLES = 1000000
BATCH = 16384

_INFO = plsc.get_sparse_core_info()
_NC, _NS, _L = _INFO.num_cores, _INFO.num_subcores, _INFO.num_lanes
_NW = _NC * _NS
_BPW = BATCH // _NW
_CHUNK = 128
_NCHUNK = _BPW // _CHUNK

_mesh = plsc.VectorSubcoreMesh(core_axis_name="c", subcore_axis_name="s")


@functools.partial(
    pl.kernel,
    mesh=_mesh,
    out_type=jax.ShapeDtypeStruct((BATCH,), jnp.float32),
    scratch_types=[
        pltpu.VMEM((_BPW,), jnp.int32),
        pltpu.VMEM((_BPW,), jnp.float32),
        pltpu.SemaphoreType.DMA,
    ],
)
def _gather_tanh(table_hbm, idx_hbm, out_hbm, idx_v, vals_v, sem):
    wid = lax.axis_index("s") * _NC + lax.axis_index("c")
    base = wid * _BPW
    pltpu.sync_copy(idx_hbm.at[pl.ds(base, _BPW)], idx_v)
    copies = []
    for j in range(_NCHUNK):
        sl = pl.ds(j * _CHUNK, _CHUNK)
        copies.append(
            pltpu.async_copy(table_hbm.at[idx_v.at[sl]], vals_v.at[sl], sem))
    for c in copies:
        c.wait()
    for i in range(_BPW // _L):
        sl = pl.ds(i * _L, _L)
        x = vals_v[sl]
        e = jnp.exp(jnp.abs(x) * -2.0)
        vals_v[sl] = jnp.sign(x) * ((1.0 - e) / (1.0 + e))
    pltpu.sync_copy(vals_v, out_hbm.at[pl.ds(base, _BPW)])


def kernel(sample_indices, sample_logits):
    return _gather_tanh(sample_logits, sample_indices)


# speedup vs baseline: 1.2163x; 1.0066x over previous
import functools

import jax
import jax.numpy as jnp
from jax import lax
from jax.experimental import pallas as pl
from jax.experimental.pallas import tpu as pltpu
from jax.experimental.pallas import tpu_sc as plsc

NUM_SAMPLES = 1000000
BATCH = 16384

_INFO = plsc.get_sparse_core_info()
_NC, _NS, _L = _INFO.num_cores, _INFO.num_subcores, _INFO.num_lanes
_NW = _NC * _NS
_BPW = BATCH // _NW
_CHUNK = 128
_NCHUNK = _BPW // _CHUNK

_mesh = plsc.VectorSubcoreMesh(core_axis_name="c", subcore_axis_name="s")


@functools.partial(
    pl.kernel,
    mesh=_mesh,
    out_type=jax.ShapeDtypeStruct((BATCH,), jnp.float32),
    scratch_types=[
        pltpu.VMEM((_BPW,), jnp.int32),
        pltpu.VMEM((_BPW,), jnp.float32),
        pltpu.SemaphoreType.DMA,
    ],
)
def _gather_tanh(table_hbm, idx_hbm, out_hbm, idx_v, vals_v, sem):
    wid = lax.axis_index("s") * _NC + lax.axis_index("c")
    base = wid * _BPW
    pltpu.sync_copy(idx_hbm.at[pl.ds(base, _BPW)], idx_v)
    copies = []
    for j in range(_NCHUNK):
        sl = pl.ds(j * _CHUNK, _CHUNK)
        copies.append(
            pltpu.async_copy(table_hbm.at[idx_v.at[sl]], vals_v.at[sl], sem))
    for c in copies:
        c.wait()

    def _tanh_step(i, _):
        sl = pl.ds(i * _L, _L)
        x = vals_v[sl]
        e = jnp.exp(jnp.abs(x) * -2.0)
        vals_v[sl] = jnp.sign(x) * ((1.0 - e) / (1.0 + e))
        return _

    lax.fori_loop(0, _BPW // _L, _tanh_step, 0)
    pltpu.sync_copy(vals_v, out_hbm.at[pl.ds(base, _BPW)])


def kernel(sample_indices, sample_logits):
    return _gather_tanh(sample_logits, sample_indices)


# speedup vs baseline: 1.2305x; 1.0117x over previous
import functools

import jax
import jax.numpy as jnp
from jax import lax
from jax.experimental import pallas as pl
from jax.experimental.pallas import tpu as pltpu
from jax.experimental.pallas import tpu_sc as plsc

NUM_SAMPLES = 1000000
BATCH = 16384

_INFO = plsc.get_sparse_core_info()
_NC, _NS, _L = _INFO.num_cores, _INFO.num_subcores, _INFO.num_lanes
_NW = _NC * _NS
_BPW = BATCH // _NW
_CHUNK = 128
_NCHUNK = _BPW // _CHUNK

_mesh = plsc.VectorSubcoreMesh(core_axis_name="c", subcore_axis_name="s")


@functools.partial(
    pl.kernel,
    mesh=_mesh,
    out_type=jax.ShapeDtypeStruct((BATCH,), jnp.float32),
    scratch_types=[
        pltpu.VMEM((_BPW,), jnp.int32),
        pltpu.VMEM((_BPW,), jnp.float32),
        pltpu.SemaphoreType.DMA,
    ],
)
def _gather_tanh(table_hbm, idx_hbm, out_hbm, idx_v, vals_v, sem):
    wid = lax.axis_index("s") * _NC + lax.axis_index("c")
    base = wid * _BPW
    pltpu.sync_copy(idx_hbm.at[pl.ds(base, _BPW)], idx_v)
    pltpu.async_copy(table_hbm.at[idx_v], vals_v, sem).wait()

    def _tanh_step(i, _):
        sl = pl.ds(i * _L, _L)
        x = vals_v[sl]
        e = jnp.exp(jnp.abs(x) * -2.0)
        vals_v[sl] = jnp.sign(x) * ((1.0 - e) / (1.0 + e))
        return _

    lax.fori_loop(0, _BPW // _L, _tanh_step, 0)
    pltpu.sync_copy(vals_v, out_hbm.at[pl.ds(base, _BPW)])


def kernel(sample_indices, sample_logits):
    return _gather_tanh(sample_logits, sample_indices)


# speedup vs baseline: 1.2335x; 1.0024x over previous
import functools

import jax
import jax.numpy as jnp
from jax import lax
from jax.experimental import pallas as pl
from jax.experimental.pallas import tpu as pltpu
from jax.experimental.pallas import tpu_sc as plsc

NUM_SAMPLES = 1000000
BATCH = 16384

_INFO = plsc.get_sparse_core_info()
_NC, _NS, _L = _INFO.num_cores, _INFO.num_subcores, _INFO.num_lanes
_NW = _NC * _NS
_BPW = BATCH // _NW
_HALF = _BPW // 2

_mesh = plsc.VectorSubcoreMesh(core_axis_name="c", subcore_axis_name="s")


@functools.partial(
    pl.kernel,
    mesh=_mesh,
    out_type=jax.ShapeDtypeStruct((BATCH,), jnp.float32),
    scratch_types=[
        pltpu.VMEM((_BPW,), jnp.int32),
        pltpu.VMEM((_BPW,), jnp.float32),
        pltpu.SemaphoreType.DMA,
        pltpu.SemaphoreType.DMA,
    ],
)
def _gather_tanh(table_hbm, idx_hbm, out_hbm, idx_v, vals_v, sem0, sem1):
    wid = lax.axis_index("s") * _NC + lax.axis_index("c")
    base = wid * _BPW
    sems = (sem0, sem1)
    idx_cp = [
        pltpu.async_copy(
            idx_hbm.at[pl.ds(base + h * _HALF, _HALF)],
            idx_v.at[pl.ds(h * _HALF, _HALF)], sems[h])
        for h in range(2)
    ]
    gathers = []
    for h in range(2):
        idx_cp[h].wait()
        gathers.append(
            pltpu.async_copy(
                table_hbm.at[idx_v.at[pl.ds(h * _HALF, _HALF)]],
                vals_v.at[pl.ds(h * _HALF, _HALF)], sems[h]))
    out_cp = []
    for h in range(2):
        gathers[h].wait()

        def _tanh_step(i, _, h=h):
            sl = pl.ds(h * _HALF + i * _L, _L)
            x = vals_v[sl]
            e = jnp.exp(jnp.abs(x) * -2.0)
            vals_v[sl] = jnp.sign(x) * ((1.0 - e) / (1.0 + e))
            return _

        lax.fori_loop(0, _HALF // _L, _tanh_step, 0)
        out_cp.append(
            pltpu.async_copy(
                vals_v.at[pl.ds(h * _HALF, _HALF)],
                out_hbm.at[pl.ds(base + h * _HALF, _HALF)], sems[h]))
    for c in out_cp:
        c.wait()


def kernel(sample_indices, sample_logits):
    return _gather_tanh(sample_logits, sample_indices)
